# Initial kernel scaffold; baseline (speedup 1.0000x reference)
import jax
import jax.numpy as jnp
from jax.experimental import pallas as pl


def kernel(user_id, amount, merchant_id, merchant_city, merchant_state, mcc, zip_2, zip_4, user_avg_amount, merchant_avg_amount, card_id, use_chip, zip_1, errors, E_user_id, E_amount, E_mer_id, E_mer_ct, E_mer_st, E_mcc, E_zip2, E_zip4, E_user_avg, E_mer_avg, W_card, b_card, W_chip, b_chip, W_zip1, b_zip1, W_hidden, b_hidden, W_out, b_out):
    raise NotImplementedError("write your pallas kernel here")


# speedup vs baseline: 2.8310x; 2.8310x over previous
import functools

import jax
import jax.numpy as jnp
from jax import lax
from jax.experimental import pallas as pl
from jax.experimental.pallas import tpu as pltpu
from jax.experimental.pallas import tpu_sc as plsc

_B = 16384
_D = 64
_DP = 128
_NT = 10
_NC = 2
_NS = 16
_NW = _NC * _NS
_RPW = _B // _NW
_CHUNK = 128
_NCHUNK = _RPW // _CHUNK

_GW = _NT * _DP
_DW = 32
_H1 = 192
_H2 = 256


def _sc_gather_body(t0, t1, t2, t3, t4, t5, t6, t7, t8, t9,
                    idx_hbm, out_hbm, idx_v, rows_v, sem):
    tables = (t0, t1, t2, t3, t4, t5, t6, t7, t8, t9)
    wid = lax.axis_index("s") * _NC + lax.axis_index("c")
    for t in range(_NT):
        def body(j, carry, _t=t):
            base = wid * _RPW + j * _CHUNK
            pltpu.sync_copy(idx_hbm.at[_t, pl.ds(base, _CHUNK)], idx_v)
            pltpu.async_copy(tables[_t].at[idx_v], rows_v, sem).wait()
            pltpu.sync_copy(rows_v,
                            out_hbm.at[pl.ds(base, _CHUNK),
                                       pl.ds(_t * _DP, _DP)])
            return carry
        lax.fori_loop(0, _NCHUNK, body, 0)


@functools.cache
def _get_sc_gather():
    return pl.kernel(
        _sc_gather_body,
        mesh=plsc.VectorSubcoreMesh(core_axis_name="c", subcore_axis_name="s"),
        out_type=jax.ShapeDtypeStruct((_B, _GW), jnp.float32),
        scratch_types=[
            pltpu.VMEM((_CHUNK,), jnp.int32),
            pltpu.VMEM((_CHUNK, _DP), jnp.float32),
            pltpu.SemaphoreType.DMA,
        ],
    )


_BLK = 1024


def _mlp_body(g_ref, d_ref, wg_ref, wd_ref, bd_ref, whd_ref, we_ref,
              bh_ref, wo_ref, bo_ref, o_ref):
    d = d_ref[...]
    a = jnp.maximum(
        jnp.dot(d, wd_ref[...], preferred_element_type=jnp.float32)
        + bd_ref[...], 0.0)
    h = jnp.dot(g_ref[...], wg_ref[...], preferred_element_type=jnp.float32)
    h = h + jnp.dot(a, whd_ref[...], preferred_element_type=jnp.float32)
    h = h + jnp.dot(d, we_ref[...], preferred_element_type=jnp.float32)
    h = jnp.maximum(h + bh_ref[...], 0.0)
    o = jnp.dot(h, wo_ref[...], preferred_element_type=jnp.float32) + bo_ref[...]
    o_ref[...] = jax.nn.sigmoid(o)


_mlp = pl.pallas_call(
    _mlp_body,
    grid=(_B // _BLK,),
    in_specs=[
        pl.BlockSpec((_BLK, _GW), lambda i: (i, 0)),
        pl.BlockSpec((_BLK, _DW), lambda i: (i, 0)),
        pl.BlockSpec((_GW, _H2), lambda i: (0, 0)),
        pl.BlockSpec((_DW, _H1), lambda i: (0, 0)),
        pl.BlockSpec((1, _H1), lambda i: (0, 0)),
        pl.BlockSpec((_H1, _H2), lambda i: (0, 0)),
        pl.BlockSpec((_DW, _H2), lambda i: (0, 0)),
        pl.BlockSpec((1, _H2), lambda i: (0, 0)),
        pl.BlockSpec((_H2, 1), lambda i: (0, 0)),
        pl.BlockSpec((1, 1), lambda i: (0, 0)),
    ],
    out_specs=pl.BlockSpec((_BLK, 1), lambda i: (i, 0)),
    out_shape=jax.ShapeDtypeStruct((_B, 1), jnp.float32),
)


def kernel(user_id, amount, merchant_id, merchant_city, merchant_state, mcc,
           zip_2, zip_4, user_avg_amount, merchant_avg_amount, card_id,
           use_chip, zip_1, errors, E_user_id, E_amount, E_mer_id, E_mer_ct,
           E_mer_st, E_mcc, E_zip2, E_zip4, E_user_avg, E_mer_avg, W_card,
           b_card, W_chip, b_chip, W_zip1, b_zip1, W_hidden, b_hidden, W_out,
           b_out):
    idx = jnp.concatenate(
        [user_id, amount, merchant_id, merchant_city, merchant_state, mcc,
         zip_2, zip_4, user_avg_amount, merchant_avg_amount],
        axis=1).astype(jnp.int32).T

    tables = [E_user_id, E_amount, E_mer_id, E_mer_ct, E_mer_st, E_mcc,
              E_zip2, E_zip4, E_user_avg, E_mer_avg]
    tables = [jnp.pad(t, ((0, 0), (0, _DP - _D))) for t in tables]

    dpack = jnp.concatenate(
        [card_id, use_chip, zip_1, errors,
         jnp.zeros((_B, _DW - 17), jnp.float32)], axis=1)

    Wh = W_hidden
    segs = [
        Wh[0:64],
        Wh[128:192],
        Wh[200:264],
        Wh[264:328],
        Wh[328:392],
        Wh[392:456] + Wh[456:520],
        Wh[648:712],
        Wh[712:776],
        Wh[776:840],
        Wh[840:904],
    ]
    zpad = jnp.zeros((_DP - _D, _H2), jnp.float32)
    wg = jnp.concatenate(
        [w for seg in segs for w in (seg, zpad)], axis=0)
    whd = jnp.concatenate([Wh[64:128], Wh[520:584], Wh[584:648]],
                          axis=0)
    we = jnp.zeros((_DW, _H2), jnp.float32).at[9:17].set(Wh[192:200])
    wd = (jnp.zeros((_DW, _H1), jnp.float32)
          .at[0:4, 0:64].set(W_card)
          .at[4:7, 64:128].set(W_chip)
          .at[7:9, 128:192].set(W_zip1))
    bd = jnp.concatenate([b_card, b_chip, b_zip1])[None, :]
    bh = b_hidden[None, :]
    bo = b_out[None, :]

    g = _get_sc_gather()(*tables, idx)

    return _mlp(g, dpack, wg, wd, bd, whd, we, bh, W_out, bo)


# speedup vs baseline: 3.3072x; 1.1682x over previous
import functools

import jax
import jax.numpy as jnp
from jax import lax
from jax.experimental import pallas as pl
from jax.experimental.pallas import tpu as pltpu
from jax.experimental.pallas import tpu_sc as plsc

_B = 16384
_D = 64
_DP = 128
_NT = 10
_NC = 2
_NS = 16
_NW = _NC * _NS
_RPW = _B // _NW
_CHUNK = 128
_NCHUNK = _RPW // _CHUNK

_GW = _NT * _DP
_DW = 32
_H1 = 192
_H2 = 256


_NBUF = 4


def _sc_gather_body(t0, t1, t2, t3, t4, t5, t6, t7, t8, t9,
                    idx_hbm, out_hbm, idx_all, b0, b1, b2, b3,
                    gs0, gs1, gs2, gs3, ws0, ws1, ws2, ws3, isem):
    tables = (t0, t1, t2, t3, t4, t5, t6, t7, t8, t9)
    bufs = (b0, b1, b2, b3)
    gsems = (gs0, gs1, gs2, gs3)
    wsems = (ws0, ws1, ws2, ws3)
    wid = lax.axis_index("s") * _NC + lax.axis_index("c")
    wbase = wid * _RPW
    pltpu.async_copy(idx_hbm.at[:, pl.ds(wbase, _RPW)], idx_all, isem).wait()

    steps = [(t, c) for t in range(_NT) for c in range(_NCHUNK)]
    gcp = [None] * len(steps)
    wcp = [None] * len(steps)

    def fire_write(s):
        t, c = steps[s]
        gcp[s].wait()
        wcp[s] = pltpu.async_copy(
            bufs[s % _NBUF],
            out_hbm.at[pl.ds(wbase + c * _CHUNK, _CHUNK),
                       pl.ds(t * _DP, _DP)],
            wsems[s % _NBUF])

    for s, (t, c) in enumerate(steps):
        if s >= _NBUF:
            wcp[s - _NBUF].wait()
        gcp[s] = pltpu.async_copy(
            tables[t].at[idx_all.at[t, pl.ds(c * _CHUNK, _CHUNK)]],
            bufs[s % _NBUF], gsems[s % _NBUF])
        if s >= 1:
            fire_write(s - 1)
    fire_write(len(steps) - 1)
    for s in range(len(steps) - _NBUF, len(steps)):
        wcp[s].wait()


@functools.cache
def _get_sc_gather():
    return pl.kernel(
        _sc_gather_body,
        mesh=plsc.VectorSubcoreMesh(core_axis_name="c", subcore_axis_name="s"),
        out_type=jax.ShapeDtypeStruct((_B, _GW), jnp.float32),
        scratch_types=[
            pltpu.VMEM((_NT, _RPW), jnp.int32),
            pltpu.VMEM((_CHUNK, _DP), jnp.float32),
            pltpu.VMEM((_CHUNK, _DP), jnp.float32),
            pltpu.VMEM((_CHUNK, _DP), jnp.float32),
            pltpu.VMEM((_CHUNK, _DP), jnp.float32),
            pltpu.SemaphoreType.DMA,
            pltpu.SemaphoreType.DMA,
            pltpu.SemaphoreType.DMA,
            pltpu.SemaphoreType.DMA,
            pltpu.SemaphoreType.DMA,
            pltpu.SemaphoreType.DMA,
            pltpu.SemaphoreType.DMA,
            pltpu.SemaphoreType.DMA,
            pltpu.SemaphoreType.DMA,
        ],
    )


_BLK = 1024


def _mlp_body(g_ref, d_ref, wg_ref, wd_ref, bd_ref, whd_ref, we_ref,
              bh_ref, wo_ref, bo_ref, o_ref):
    d = d_ref[...]
    a = jnp.maximum(
        jnp.dot(d, wd_ref[...], preferred_element_type=jnp.float32)
        + bd_ref[...], 0.0)
    h = jnp.dot(g_ref[...], wg_ref[...], preferred_element_type=jnp.float32)
    h = h + jnp.dot(a, whd_ref[...], preferred_element_type=jnp.float32)
    h = h + jnp.dot(d, we_ref[...], preferred_element_type=jnp.float32)
    h = jnp.maximum(h + bh_ref[...], 0.0)
    o = jnp.dot(h, wo_ref[...], preferred_element_type=jnp.float32) + bo_ref[...]
    o_ref[...] = jax.nn.sigmoid(o)


_mlp = pl.pallas_call(
    _mlp_body,
    grid=(_B // _BLK,),
    in_specs=[
        pl.BlockSpec((_BLK, _GW), lambda i: (i, 0)),
        pl.BlockSpec((_BLK, _DW), lambda i: (i, 0)),
        pl.BlockSpec((_GW, _H2), lambda i: (0, 0)),
        pl.BlockSpec((_DW, _H1), lambda i: (0, 0)),
        pl.BlockSpec((1, _H1), lambda i: (0, 0)),
        pl.BlockSpec((_H1, _H2), lambda i: (0, 0)),
        pl.BlockSpec((_DW, _H2), lambda i: (0, 0)),
        pl.BlockSpec((1, _H2), lambda i: (0, 0)),
        pl.BlockSpec((_H2, 1), lambda i: (0, 0)),
        pl.BlockSpec((1, 1), lambda i: (0, 0)),
    ],
    out_specs=pl.BlockSpec((_BLK, 1), lambda i: (i, 0)),
    out_shape=jax.ShapeDtypeStruct((_B, 1), jnp.float32),
)


def kernel(user_id, amount, merchant_id, merchant_city, merchant_state, mcc,
           zip_2, zip_4, user_avg_amount, merchant_avg_amount, card_id,
           use_chip, zip_1, errors, E_user_id, E_amount, E_mer_id, E_mer_ct,
           E_mer_st, E_mcc, E_zip2, E_zip4, E_user_avg, E_mer_avg, W_card,
           b_card, W_chip, b_chip, W_zip1, b_zip1, W_hidden, b_hidden, W_out,
           b_out):
    idx = jnp.concatenate(
        [user_id, amount, merchant_id, merchant_city, merchant_state, mcc,
         zip_2, zip_4, user_avg_amount, merchant_avg_amount],
        axis=1).astype(jnp.int32).T

    tables = [E_user_id, E_amount, E_mer_id, E_mer_ct, E_mer_st, E_mcc,
              E_zip2, E_zip4, E_user_avg, E_mer_avg]
    tables = [jnp.pad(t, ((0, 0), (0, _DP - _D))) for t in tables]

    dpack = jnp.concatenate(
        [card_id, use_chip, zip_1, errors,
         jnp.zeros((_B, _DW - 17), jnp.float32)], axis=1)

    Wh = W_hidden
    segs = [
        Wh[0:64],
        Wh[128:192],
        Wh[200:264],
        Wh[264:328],
        Wh[328:392],
        Wh[392:456] + Wh[456:520],
        Wh[648:712],
        Wh[712:776],
        Wh[776:840],
        Wh[840:904],
    ]
    zpad = jnp.zeros((_DP - _D, _H2), jnp.float32)
    wg = jnp.concatenate(
        [w for seg in segs for w in (seg, zpad)], axis=0)
    whd = jnp.concatenate([Wh[64:128], Wh[520:584], Wh[584:648]],
                          axis=0)
    we = jnp.zeros((_DW, _H2), jnp.float32).at[9:17].set(Wh[192:200])
    wd = (jnp.zeros((_DW, _H1), jnp.float32)
          .at[0:4, 0:64].set(W_card)
          .at[4:7, 64:128].set(W_chip)
          .at[7:9, 128:192].set(W_zip1))
    bd = jnp.concatenate([b_card, b_chip, b_zip1])[None, :]
    bh = b_hidden[None, :]
    bo = b_out[None, :]

    g = _get_sc_gather()(*tables, idx)

    return _mlp(g, dpack, wg, wd, bd, whd, we, bh, W_out, bo)


# speedup vs baseline: 3.4585x; 1.0457x over previous
import functools

import jax
import jax.numpy as jnp
from jax import lax
from jax.experimental import pallas as pl
from jax.experimental.pallas import tpu as pltpu
from jax.experimental.pallas import tpu_sc as plsc

_B = 16384
_D = 64
_DP = 128
_NT = 10
_NC = 2
_NS = 16
_NW = _NC * _NS
_CHUNK = 128
_NBUF = 4

_GW = _NT * _DP
_DW = 32
_H1 = 192
_H2 = 256

_NSPLIT = 2
_S = _B // _NSPLIT


def _sc_gather_body(rpw, t0, t1, t2, t3, t4, t5, t6, t7, t8, t9,
                    idx_hbm, out_hbm, idx_all, b0, b1, b2, b3,
                    gs0, gs1, gs2, gs3, ws0, ws1, ws2, ws3, isem):
    tables = (t0, t1, t2, t3, t4, t5, t6, t7, t8, t9)
    bufs = (b0, b1, b2, b3)
    gsems = (gs0, gs1, gs2, gs3)
    wsems = (ws0, ws1, ws2, ws3)
    nchunk = rpw // _CHUNK
    wid = lax.axis_index("s") * _NC + lax.axis_index("c")
    wbase = wid * rpw
    pltpu.async_copy(idx_hbm.at[:, pl.ds(wbase, rpw)], idx_all, isem).wait()

    steps = [(t, c) for t in range(_NT) for c in range(nchunk)]
    gcp = [None] * len(steps)
    wcp = [None] * len(steps)

    def fire_write(s):
        t, c = steps[s]
        gcp[s].wait()
        wcp[s] = pltpu.async_copy(
            bufs[s % _NBUF],
            out_hbm.at[pl.ds(wbase + c * _CHUNK, _CHUNK),
                       pl.ds(t * _DP, _DP)],
            wsems[s % _NBUF])

    for s, (t, c) in enumerate(steps):
        if s >= _NBUF:
            wcp[s - _NBUF].wait()
        gcp[s] = pltpu.async_copy(
            tables[t].at[idx_all.at[t, pl.ds(c * _CHUNK, _CHUNK)]],
            bufs[s % _NBUF], gsems[s % _NBUF])
        if s >= 1:
            fire_write(s - 1)
    fire_write(len(steps) - 1)
    for s in range(max(0, len(steps) - _NBUF), len(steps)):
        wcp[s].wait()


@functools.cache
def _get_sc_gather(rows):
    rpw = rows // _NW
    return pl.kernel(
        functools.partial(_sc_gather_body, rpw),
        mesh=plsc.VectorSubcoreMesh(core_axis_name="c", subcore_axis_name="s"),
        out_type=jax.ShapeDtypeStruct((rows, _GW), jnp.float32),
        scratch_types=[
            pltpu.VMEM((_NT, rpw), jnp.int32),
            pltpu.VMEM((_CHUNK, _DP), jnp.float32),
            pltpu.VMEM((_CHUNK, _DP), jnp.float32),
            pltpu.VMEM((_CHUNK, _DP), jnp.float32),
            pltpu.VMEM((_CHUNK, _DP), jnp.float32),
            pltpu.SemaphoreType.DMA,
            pltpu.SemaphoreType.DMA,
            pltpu.SemaphoreType.DMA,
            pltpu.SemaphoreType.DMA,
            pltpu.SemaphoreType.DMA,
            pltpu.SemaphoreType.DMA,
            pltpu.SemaphoreType.DMA,
            pltpu.SemaphoreType.DMA,
            pltpu.SemaphoreType.DMA,
        ],
    )


_BLK = 1024


def _mlp_body(g_ref, d_ref, wg_ref, wd_ref, bd_ref, whd_ref, we_ref,
              bh_ref, wo_ref, bo_ref, o_ref):
    d = d_ref[...]
    a = jnp.maximum(
        jnp.dot(d, wd_ref[...], preferred_element_type=jnp.float32)
        + bd_ref[...], 0.0)
    h = jnp.dot(g_ref[...], wg_ref[...], preferred_element_type=jnp.float32)
    h = h + jnp.dot(a, whd_ref[...], preferred_element_type=jnp.float32)
    h = h + jnp.dot(d, we_ref[...], preferred_element_type=jnp.float32)
    h = jnp.maximum(h + bh_ref[...], 0.0)
    o = jnp.dot(h, wo_ref[...], preferred_element_type=jnp.float32) + bo_ref[...]
    o_ref[...] = jax.nn.sigmoid(o)


@functools.cache
def _get_mlp(rows):
    return pl.pallas_call(
        _mlp_body,
        grid=(rows // _BLK,),
        in_specs=[
            pl.BlockSpec((_BLK, _GW), lambda i: (i, 0)),
            pl.BlockSpec((_BLK, _DW), lambda i: (i, 0)),
            pl.BlockSpec((_GW, _H2), lambda i: (0, 0)),
            pl.BlockSpec((_DW, _H1), lambda i: (0, 0)),
            pl.BlockSpec((1, _H1), lambda i: (0, 0)),
            pl.BlockSpec((_H1, _H2), lambda i: (0, 0)),
            pl.BlockSpec((_DW, _H2), lambda i: (0, 0)),
            pl.BlockSpec((1, _H2), lambda i: (0, 0)),
            pl.BlockSpec((_H2, 1), lambda i: (0, 0)),
            pl.BlockSpec((1, 1), lambda i: (0, 0)),
        ],
        out_specs=pl.BlockSpec((_BLK, 1), lambda i: (i, 0)),
        out_shape=jax.ShapeDtypeStruct((rows, 1), jnp.float32),
    )


def kernel(user_id, amount, merchant_id, merchant_city, merchant_state, mcc,
           zip_2, zip_4, user_avg_amount, merchant_avg_amount, card_id,
           use_chip, zip_1, errors, E_user_id, E_amount, E_mer_id, E_mer_ct,
           E_mer_st, E_mcc, E_zip2, E_zip4, E_user_avg, E_mer_avg, W_card,
           b_card, W_chip, b_chip, W_zip1, b_zip1, W_hidden, b_hidden, W_out,
           b_out):
    idx = jnp.concatenate(
        [user_id, amount, merchant_id, merchant_city, merchant_state, mcc,
         zip_2, zip_4, user_avg_amount, merchant_avg_amount],
        axis=1).astype(jnp.int32).T

    tables = [E_user_id, E_amount, E_mer_id, E_mer_ct, E_mer_st, E_mcc,
              E_zip2, E_zip4, E_user_avg, E_mer_avg]
    tables = [jnp.pad(t, ((0, 0), (0, _DP - _D))) for t in tables]

    dpack = jnp.concatenate(
        [card_id, use_chip, zip_1, errors,
         jnp.zeros((_B, _DW - 17), jnp.float32)], axis=1)

    Wh = W_hidden
    segs = [
        Wh[0:64],
        Wh[128:192],
        Wh[200:264],
        Wh[264:328],
        Wh[328:392],
        Wh[392:456] + Wh[456:520],
        Wh[648:712],
        Wh[712:776],
        Wh[776:840],
        Wh[840:904],
    ]
    zpad = jnp.zeros((_DP - _D, _H2), jnp.float32)
    wg = jnp.concatenate(
        [w for seg in segs for w in (seg, zpad)], axis=0)
    whd = jnp.concatenate([Wh[64:128], Wh[520:584], Wh[584:648]],
                          axis=0)
    we = jnp.zeros((_DW, _H2), jnp.float32).at[9:17].set(Wh[192:200])
    wd = (jnp.zeros((_DW, _H1), jnp.float32)
          .at[0:4, 0:64].set(W_card)
          .at[4:7, 64:128].set(W_chip)
          .at[7:9, 128:192].set(W_zip1))
    bd = jnp.concatenate([b_card, b_chip, b_zip1])[None, :]
    bh = b_hidden[None, :]
    bo = b_out[None, :]

    sc = _get_sc_gather(_S)
    mlp = _get_mlp(_S)
    outs = []
    for i in range(_NSPLIT):
        g = sc(*tables, lax.slice(idx, (0, i * _S), (_NT, (i + 1) * _S)))
        outs.append(mlp(g, lax.slice(dpack, (i * _S, 0), ((i + 1) * _S, _DW)),
                        wg, wd, bd, whd, we, bh, W_out, bo))
    return jnp.concatenate(outs, axis=0)


# speedup vs baseline: 3.4755x; 1.0049x over previous
import functools

import jax
import jax.numpy as jnp
from jax import lax
from jax.experimental import pallas as pl
from jax.experimental.pallas import tpu as pltpu
from jax.experimental.pallas import tpu_sc as plsc

_B = 16384
_D = 64
_DP = 128
_NT = 10
_NC = 2
_NS = 16
_NW = _NC * _NS
_CHUNK = 128
_NBUF = 4

_GW = _NT * _DP
_DW = 32
_H1 = 192
_H2 = 256

_NSPLIT = 4
_S = _B // _NSPLIT


def _sc_gather_body(rpw, t0, t1, t2, t3, t4, t5, t6, t7, t8, t9,
                    idx_hbm, out_hbm, idx_all, b0, b1, b2, b3,
                    gs0, gs1, gs2, gs3, ws0, ws1, ws2, ws3, isem):
    tables = (t0, t1, t2, t3, t4, t5, t6, t7, t8, t9)
    bufs = (b0, b1, b2, b3)
    gsems = (gs0, gs1, gs2, gs3)
    wsems = (ws0, ws1, ws2, ws3)
    nchunk = rpw // _CHUNK
    wid = lax.axis_index("s") * _NC + lax.axis_index("c")
    wbase = wid * rpw
    pltpu.async_copy(idx_hbm.at[:, pl.ds(wbase, rpw)], idx_all, isem).wait()

    steps = [(t, c) for t in range(_NT) for c in range(nchunk)]
    gcp = [None] * len(steps)
    wcp = [None] * len(steps)

    def fire_write(s):
        t, c = steps[s]
        gcp[s].wait()
        wcp[s] = pltpu.async_copy(
            bufs[s % _NBUF],
            out_hbm.at[pl.ds(wbase + c * _CHUNK, _CHUNK),
                       pl.ds(t * _DP, _DP)],
            wsems[s % _NBUF])

    for s, (t, c) in enumerate(steps):
        if s >= _NBUF:
            wcp[s - _NBUF].wait()
        gcp[s] = pltpu.async_copy(
            tables[t].at[idx_all.at[t, pl.ds(c * _CHUNK, _CHUNK)]],
            bufs[s % _NBUF], gsems[s % _NBUF])
        if s >= 1:
            fire_write(s - 1)
    fire_write(len(steps) - 1)
    for s in range(max(0, len(steps) - _NBUF), len(steps)):
        wcp[s].wait()


@functools.cache
def _get_sc_gather(rows):
    rpw = rows // _NW
    return pl.kernel(
        functools.partial(_sc_gather_body, rpw),
        mesh=plsc.VectorSubcoreMesh(core_axis_name="c", subcore_axis_name="s"),
        out_type=jax.ShapeDtypeStruct((rows, _GW), jnp.float32),
        scratch_types=[
            pltpu.VMEM((_NT, rpw), jnp.int32),
            pltpu.VMEM((_CHUNK, _DP), jnp.float32),
            pltpu.VMEM((_CHUNK, _DP), jnp.float32),
            pltpu.VMEM((_CHUNK, _DP), jnp.float32),
            pltpu.VMEM((_CHUNK, _DP), jnp.float32),
            pltpu.SemaphoreType.DMA,
            pltpu.SemaphoreType.DMA,
            pltpu.SemaphoreType.DMA,
            pltpu.SemaphoreType.DMA,
            pltpu.SemaphoreType.DMA,
            pltpu.SemaphoreType.DMA,
            pltpu.SemaphoreType.DMA,
            pltpu.SemaphoreType.DMA,
            pltpu.SemaphoreType.DMA,
        ],
    )


_BLK = 1024


def _mlp_body(g_ref, d_ref, wg_ref, wd_ref, bd_ref, whd_ref, we_ref,
              bh_ref, wo_ref, bo_ref, o_ref):
    d = d_ref[...]
    a = jnp.maximum(
        jnp.dot(d, wd_ref[...], preferred_element_type=jnp.float32)
        + bd_ref[...], 0.0)
    h = jnp.dot(g_ref[...], wg_ref[...], preferred_element_type=jnp.float32)
    h = h + jnp.dot(a, whd_ref[...], preferred_element_type=jnp.float32)
    h = h + jnp.dot(d, we_ref[...], preferred_element_type=jnp.float32)
    h = jnp.maximum(h + bh_ref[...], 0.0)
    o = jnp.dot(h, wo_ref[...], preferred_element_type=jnp.float32) + bo_ref[...]
    o_ref[...] = jax.nn.sigmoid(o)


@functools.cache
def _get_mlp(rows):
    return pl.pallas_call(
        _mlp_body,
        grid=(rows // _BLK,),
        in_specs=[
            pl.BlockSpec((_BLK, _GW), lambda i: (i, 0)),
            pl.BlockSpec((_BLK, _DW), lambda i: (i, 0)),
            pl.BlockSpec((_GW, _H2), lambda i: (0, 0)),
            pl.BlockSpec((_DW, _H1), lambda i: (0, 0)),
            pl.BlockSpec((1, _H1), lambda i: (0, 0)),
            pl.BlockSpec((_H1, _H2), lambda i: (0, 0)),
            pl.BlockSpec((_DW, _H2), lambda i: (0, 0)),
            pl.BlockSpec((1, _H2), lambda i: (0, 0)),
            pl.BlockSpec((_H2, 1), lambda i: (0, 0)),
            pl.BlockSpec((1, 1), lambda i: (0, 0)),
        ],
        out_specs=pl.BlockSpec((_BLK, 1), lambda i: (i, 0)),
        out_shape=jax.ShapeDtypeStruct((rows, 1), jnp.float32),
    )


def kernel(user_id, amount, merchant_id, merchant_city, merchant_state, mcc,
           zip_2, zip_4, user_avg_amount, merchant_avg_amount, card_id,
           use_chip, zip_1, errors, E_user_id, E_amount, E_mer_id, E_mer_ct,
           E_mer_st, E_mcc, E_zip2, E_zip4, E_user_avg, E_mer_avg, W_card,
           b_card, W_chip, b_chip, W_zip1, b_zip1, W_hidden, b_hidden, W_out,
           b_out):
    idx = jnp.concatenate(
        [user_id, amount, merchant_id, merchant_city, merchant_state, mcc,
         zip_2, zip_4, user_avg_amount, merchant_avg_amount],
        axis=1).astype(jnp.int32).T

    tables = [E_user_id, E_amount, E_mer_id, E_mer_ct, E_mer_st, E_mcc,
              E_zip2, E_zip4, E_user_avg, E_mer_avg]
    tables = [jnp.pad(t, ((0, 0), (0, _DP - _D))) for t in tables]

    dpack = jnp.concatenate(
        [card_id, use_chip, zip_1, errors,
         jnp.zeros((_B, _DW - 17), jnp.float32)], axis=1)

    Wh = W_hidden
    segs = [
        Wh[0:64],
        Wh[128:192],
        Wh[200:264],
        Wh[264:328],
        Wh[328:392],
        Wh[392:456] + Wh[456:520],
        Wh[648:712],
        Wh[712:776],
        Wh[776:840],
        Wh[840:904],
    ]
    zpad = jnp.zeros((_DP - _D, _H2), jnp.float32)
    wg = jnp.concatenate(
        [w for seg in segs for w in (seg, zpad)], axis=0)
    whd = jnp.concatenate([Wh[64:128], Wh[520:584], Wh[584:648]],
                          axis=0)
    we = jnp.zeros((_DW, _H2), jnp.float32).at[9:17].set(Wh[192:200])
    wd = (jnp.zeros((_DW, _H1), jnp.float32)
          .at[0:4, 0:64].set(W_card)
          .at[4:7, 64:128].set(W_chip)
          .at[7:9, 128:192].set(W_zip1))
    bd = jnp.concatenate([b_card, b_chip, b_zip1])[None, :]
    bh = b_hidden[None, :]
    bo = b_out[None, :]

    sc = _get_sc_gather(_S)
    mlp = _get_mlp(_S)
    outs = []
    for i in range(_NSPLIT):
        g = sc(*tables, lax.slice(idx, (0, i * _S), (_NT, (i + 1) * _S)))
        outs.append(mlp(g, lax.slice(dpack, (i * _S, 0), ((i + 1) * _S, _DW)),
                        wg, wd, bd, whd, we, bh, W_out, bo))
    return jnp.concatenate(outs, axis=0)
